# baseline (speedup 1.0000x reference)
import jax
import jax.numpy as jnp
from jax import lax
from jax.experimental import pallas as pl
from jax.experimental.pallas import tpu as pltpu

N_ATOMS = 32768
D = 128
HID = 32
NH = 3
C = 16
J = C * NH
BLK = 8192
NB = N_ATOMS // BLK
NEG = -1e30
RSPLIT = 8


def _split_reduce(x, op):
    q = x.shape[0] // RSPLIT
    parts = [x[k * q:(k + 1) * q] for k in range(RSPLIT)]
    while len(parts) > 1:
        parts = [op(parts[i], parts[i + 1]) for i in range(0, len(parts), 2)]
    return parts[0]


def _tc_body(owner_ref, fea_ref, w1_ref, b1_ref, w2_ref,
             out_ref, m_ref, s_ref, acc_ref, w2e_ref, colc_ref):
    b = pl.program_id(0)

    @pl.when(b == 0)
    def _prep():
        for c in range(C):
            w2e_ref[:, c * NH:(c + 1) * NH] = w2_ref[...]
        colc_ref[...] = lax.broadcasted_iota(jnp.int32, (1, J), 1) // NH

    fea = fea_ref[...]
    h1 = jnp.dot(fea, w1_ref[...], preferred_element_type=jnp.float32)
    h1 = h1 + b1_ref[0, :]
    h1 = h1 * jax.nn.sigmoid(h1)
    w48 = jnp.dot(h1, w2e_ref[...], preferred_element_type=jnp.float32)

    owner = owner_ref[0, 0, :]
    onehot = colc_ref[0, :] == owner[:, None]

    masked = jnp.where(onehot, w48, NEG)
    m_b = _split_reduce(masked, jnp.maximum)
    m_b = jnp.max(m_b, axis=0)[None, :]
    e = jnp.exp(masked - m_b)
    s_b = _split_reduce(e, jnp.add)
    s_b = jnp.sum(s_b, axis=0)[None, :]
    accT_b = lax.dot_general(fea, e, (((0,), (0,)), ((), ())),
                             preferred_element_type=jnp.float32)

    @pl.when(b == 0)
    def _init():
        m_ref[...] = m_b
        s_ref[...] = s_b
        acc_ref[...] = accT_b

    @pl.when(b > 0)
    def _update():
        m_old = m_ref[...]
        m_new = jnp.maximum(m_old, m_b)
        a_old = jnp.exp(m_old - m_new)
        a_new = jnp.exp(m_b - m_new)
        m_ref[...] = m_new
        s_ref[...] = a_old * s_ref[...] + a_new * s_b
        acc_ref[...] = a_old * acc_ref[...] + a_new * accT_b

    @pl.when(b == NB - 1)
    def _finish():
        s = s_ref[...]
        denom = jnp.where(s > 0.0, s, 1.0)
        nonempty = m_ref[...] > (NEG * 0.5)
        out_ref[...] = jnp.where(nonempty, acc_ref[...] / denom, 0.0)


def kernel(atom_feas, atom_owner, W1, b1, W2, b2):
    owner3 = atom_owner.astype(jnp.int32).reshape(NB, 1, BLK)
    b1r = b1.reshape(1, HID)

    outT = pl.pallas_call(
        _tc_body,
        grid=(NB,),
        in_specs=[
            pl.BlockSpec((1, 1, BLK), lambda b: (b, 0, 0)),
            pl.BlockSpec((BLK, D), lambda b: (b, 0)),
            pl.BlockSpec((D, HID), lambda b: (0, 0)),
            pl.BlockSpec((1, HID), lambda b: (0, 0)),
            pl.BlockSpec((HID, NH), lambda b: (0, 0)),
        ],
        out_specs=pl.BlockSpec((D, J), lambda b: (0, 0)),
        out_shape=jax.ShapeDtypeStruct((D, J), jnp.float32),
        scratch_shapes=[
            pltpu.VMEM((1, J), jnp.float32),
            pltpu.VMEM((1, J), jnp.float32),
            pltpu.VMEM((D, J), jnp.float32),
            pltpu.VMEM((HID, J), jnp.float32),
            pltpu.VMEM((1, J), jnp.int32),
        ],
        compiler_params=pltpu.CompilerParams(
            dimension_semantics=("arbitrary",),
        ),
    )(owner3, atom_feas, W1, b1r, W2)

    return outT.reshape(D, C, NH).transpose(1, 0, 2).reshape(C, D * NH)


# speedup vs baseline: 1.0546x; 1.0546x over previous
import jax
import jax.numpy as jnp
from jax import lax
from jax.experimental import pallas as pl
from jax.experimental.pallas import tpu as pltpu

N_ATOMS = 32768
D = 128
HID = 32
NH = 3
C = 16
J = C * NH
BLK = 8192
NB = N_ATOMS // BLK
NEG = -1e30
RSPLIT = 8


def _split_reduce(x, op):
    q = x.shape[0] // RSPLIT
    parts = [x[k * q:(k + 1) * q] for k in range(RSPLIT)]
    while len(parts) > 1:
        parts = [op(parts[i], parts[i + 1]) for i in range(0, len(parts), 2)]
    return parts[0]


def _tc_body(owner_ref, fea_ref, w1_ref, b1_ref, w2_ref,
             out_ref, m_ref, s_ref, acc_ref, w2e_ref, colc_ref):
    b = pl.program_id(0)

    @pl.when(b == 0)
    def _prep():
        for c in range(C):
            w2e_ref[:, c * NH:(c + 1) * NH] = w2_ref[...]
        colc_ref[...] = lax.broadcasted_iota(jnp.int32, (1, J), 1) // NH

    fea = fea_ref[...]
    h1 = jnp.dot(fea, w1_ref[...], preferred_element_type=jnp.float32)
    h1 = h1 + b1_ref[0, :]
    h1 = h1 * jax.nn.sigmoid(h1)
    w48 = jnp.dot(h1, w2e_ref[...], preferred_element_type=jnp.float32)

    owner = owner_ref[0, 0, :]
    onehot = colc_ref[0, :] == owner[:, None]

    masked = jnp.where(onehot, w48, NEG)
    m_b = _split_reduce(masked, jnp.maximum)
    m_b = jnp.max(m_b, axis=0)[None, :]
    e = jnp.exp(masked - m_b)
    s_b = _split_reduce(e, jnp.add)
    s_b = jnp.sum(s_b, axis=0)[None, :]
    accT_b = lax.dot_general(fea, e, (((0,), (0,)), ((), ())),
                             preferred_element_type=jnp.float32)

    @pl.when(b == 0)
    def _init():
        m_ref[...] = m_b
        s_ref[...] = s_b
        acc_ref[...] = accT_b

    @pl.when(b > 0)
    def _update():
        m_old = m_ref[...]
        m_new = jnp.maximum(m_old, m_b)
        a_old = jnp.exp(m_old - m_new)
        a_new = jnp.exp(m_b - m_new)
        m_ref[...] = m_new
        s_ref[...] = a_old * s_ref[...] + a_new * s_b
        acc_ref[...] = a_old * acc_ref[...] + a_new * accT_b

    @pl.when(b == NB - 1)
    def _finish():
        s = s_ref[...]
        denom = jnp.where(s > 0.0, s, 1.0)
        nonempty = m_ref[...] > (NEG * 0.5)
        normed = jnp.where(nonempty, acc_ref[...] / denom, 0.0)
        out_ref[...] = normed.reshape(D, C, NH).transpose(1, 0, 2).reshape(
            C, D * NH)


def kernel(atom_feas, atom_owner, W1, b1, W2, b2):
    owner3 = atom_owner.astype(jnp.int32).reshape(NB, 1, BLK)
    b1r = b1.reshape(1, HID)

    out = pl.pallas_call(
        _tc_body,
        grid=(NB,),
        in_specs=[
            pl.BlockSpec((1, 1, BLK), lambda b: (b, 0, 0)),
            pl.BlockSpec((BLK, D), lambda b: (b, 0)),
            pl.BlockSpec((D, HID), lambda b: (0, 0)),
            pl.BlockSpec((1, HID), lambda b: (0, 0)),
            pl.BlockSpec((HID, NH), lambda b: (0, 0)),
        ],
        out_specs=pl.BlockSpec((C, D * NH), lambda b: (0, 0)),
        out_shape=jax.ShapeDtypeStruct((C, D * NH), jnp.float32),
        scratch_shapes=[
            pltpu.VMEM((1, J), jnp.float32),
            pltpu.VMEM((1, J), jnp.float32),
            pltpu.VMEM((D, J), jnp.float32),
            pltpu.VMEM((HID, J), jnp.float32),
            pltpu.VMEM((1, J), jnp.int32),
        ],
        compiler_params=pltpu.CompilerParams(
            dimension_semantics=("arbitrary",),
        ),
    )(owner3, atom_feas, W1, b1r, W2)
    return out
